# baseline (speedup 1.0000x reference)
import functools

import jax
import jax.numpy as jnp
from jax import lax
from jax.experimental import pallas as pl
from jax.experimental.pallas import tpu as pltpu
from jax.experimental.pallas import tpu_sc as plsc

B = 4096
F = 26
V = 100000
D = 16
DENSE = 13
NF = F + 1
FD = F * D
EPS = 1e-5

NC = 2
NS = 16
NW = NC * NS
ROWS = B * F
RPW = ROWS // NW
CH = 128
NCH = RPW // CH


def _sc_gather_body(flat_ref, flatq_ref, emb_ref, lin16_ref, eout_ref, lout_ref,
                    idx_v, qidx_v, erows_v, l16_v, sem_e, sem_l):
    wid = lax.axis_index("s") * NC + lax.axis_index("c")
    r0 = wid * RPW
    pltpu.sync_copy(flat_ref.at[pl.ds(r0, RPW)], idx_v)
    pltpu.sync_copy(flatq_ref.at[pl.ds(r0, RPW)], qidx_v)

    def fire(j, carry):
        o = j * CH
        pltpu.async_copy(emb_ref.at[idx_v.at[pl.ds(o, CH)]],
                         erows_v.at[pl.ds(o, CH)], sem_e)
        pltpu.async_copy(lin16_ref.at[qidx_v.at[pl.ds(o, CH)]],
                         l16_v.at[pl.ds(o, CH)], sem_l)
        return carry

    lax.fori_loop(0, NCH, fire, 0)

    def drain(j, carry):
        o = j * CH
        pltpu.make_async_copy(emb_ref.at[idx_v.at[pl.ds(o, CH)]],
                              erows_v.at[pl.ds(o, CH)], sem_e).wait()
        pltpu.make_async_copy(lin16_ref.at[qidx_v.at[pl.ds(o, CH)]],
                              l16_v.at[pl.ds(o, CH)], sem_l).wait()
        return carry

    lax.fori_loop(0, NCH, drain, 0)
    pltpu.sync_copy(erows_v, eout_ref.at[pl.ds(r0, RPW)])
    pltpu.sync_copy(l16_v, lout_ref.at[pl.ds(r0, RPW)])


def _sc_gather(flat, flatq, emb_table, lin16):
    mesh = plsc.VectorSubcoreMesh(core_axis_name="c", subcore_axis_name="s")
    k = pl.kernel(
        _sc_gather_body,
        out_type=(
            jax.ShapeDtypeStruct((ROWS, D), jnp.float32),
            jax.ShapeDtypeStruct((ROWS, D), jnp.float32),
        ),
        mesh=mesh,
        compiler_params=pltpu.CompilerParams(use_tc_tiling_on_sc=False),
        scratch_types=[
            pltpu.VMEM((RPW,), jnp.int32),
            pltpu.VMEM((RPW,), jnp.int32),
            pltpu.VMEM((RPW, D), jnp.float32),
            pltpu.VMEM((RPW, D), jnp.float32),
            pltpu.SemaphoreType.DMA,
            pltpu.SemaphoreType.DMA,
        ],
    )
    return k(flat, flatq, emb_table, lin16)


BT = 256


def _tc_body(ge_ref, gl16_ref, m_ref, dx_ref, wd_ref, bd_ref, wdl_ref, a_ref,
             r_ref, w1a_ref, w1b_ref, b1_ref, w2_ref, b2_ref, w3_ref, c0_ref,
             out_ref):
    ge = ge_ref[...]
    dx = dx_ref[...]
    de = jnp.dot(dx, wd_ref[...], preferred_element_type=jnp.float32) + bd_ref[...]
    s = jnp.dot(ge, a_ref[...], preferred_element_type=jnp.float32) + de
    sumsq = (jnp.sum(ge * ge, axis=1, keepdims=True)
             + jnp.sum(de * de, axis=1, keepdims=True))
    order2 = jnp.sum(s * s, axis=1, keepdims=True) - sumsq
    m_exp = jnp.dot(m_ref[...], r_ref[...], preferred_element_type=jnp.float32)
    lane = jnp.bitwise_and(
        lax.broadcasted_iota(jnp.int32, (BT, FD), 1), 15).astype(jnp.float32)
    lsel = jnp.where(m_exp == lane, gl16_ref[...], 0.0)
    order1 = (jnp.sum(lsel, axis=1, keepdims=True)
              + jnp.sum(dx * wdl_ref[...], axis=1, keepdims=True))
    h = (jnp.dot(ge, w1a_ref[...], preferred_element_type=jnp.float32)
         + jnp.dot(de, w1b_ref[...], preferred_element_type=jnp.float32)
         + b1_ref[...])
    h = jnp.where(h >= 0, h, 0.2 * h)
    h = jnp.dot(h, w2_ref[...], preferred_element_type=jnp.float32) + b2_ref[...]
    h = jnp.where(h >= 0, h, 0.2 * h)
    z = jnp.sum(h * w3_ref[...], axis=1, keepdims=True)
    out_ref[...] = z + order1 + 0.5 * order2 + c0_ref[...]


def _tc_dense(ge, gl16, m, dx, wd, bd, wdl, a, r, w1a, w1b, b1p, w2p, b2p,
              w3p, c0):
    full = lambda shape: pl.BlockSpec(shape, lambda i: (0,) * len(shape))
    return pl.pallas_call(
        _tc_body,
        grid=(B // BT,),
        in_specs=[
            pl.BlockSpec((BT, FD), lambda i: (i, 0)),
            pl.BlockSpec((BT, FD), lambda i: (i, 0)),
            pl.BlockSpec((BT, F), lambda i: (i, 0)),
            pl.BlockSpec((BT, DENSE), lambda i: (i, 0)),
            full((DENSE, D)),
            full((1, D)),
            full((1, DENSE)),
            full((FD, D)),
            full((F, FD)),
            full((FD, 256)),
            full((D, 256)),
            full((1, 256)),
            full((256, 128)),
            full((1, 128)),
            full((1, 128)),
            full((1, 1)),
        ],
        out_specs=pl.BlockSpec((BT, 1), lambda i: (i, 0)),
        out_shape=jax.ShapeDtypeStruct((B, 1), jnp.float32),
    )(ge, gl16, m, dx, wd, bd, wdl, a, r, w1a, w1b, b1p, w2p, b2p, w3p, c0)


def kernel(ids, sparse_xs, dense_xs, emb_table, lin_table, Wd, bd, Wdl, bdl,
           bn0_g, bn0_b, W1, b1, bn1_g, bn1_b, W2, b2, bn2_g, bn2_b, W3, b3,
           global_bias):
    offsets = jnp.arange(F, dtype=ids.dtype) * V
    flat = (ids + offsets[None, :]).reshape(ROWS).astype(jnp.int32)
    flatq = flat >> 4
    lin16 = lin_table.reshape(F * V // 16, 16)

    inv = 1.0 / jnp.sqrt(1.0 + EPS)
    f_idx = jnp.arange(FD) // D
    d_idx = jnp.arange(FD) % D
    perm_sparse = d_idx * NF + f_idx
    perm_dense = jnp.arange(D) * NF + F
    w1s = (bn0_g * inv)[:, None] * W1
    w1a = w1s[perm_sparse]
    w1b = w1s[perm_dense]
    b1p = (b1 + bn0_b @ W1)[None, :]
    w2p = (bn1_g * inv)[:, None] * W2
    b2p = (b2 + bn1_b @ W2)[None, :]
    w3p = ((bn2_g * inv) * W3[:, 0])[None, :]
    c0 = (b3[0] + bn2_b @ W3[:, 0] + global_bias[0] + bdl[0]).reshape(1, 1)
    a = jnp.tile(jnp.eye(D, dtype=jnp.float32), (F, 1))
    r = jnp.repeat(jnp.eye(F, dtype=jnp.float32), D, axis=1)
    m = jnp.bitwise_and(flat, 15).astype(jnp.float32).reshape(B, F)

    erows, lrows = _sc_gather(flat, flatq, emb_table, lin16)
    ge = erows.reshape(B, FD)
    gl16 = lrows.reshape(B, FD)

    return _tc_dense(ge, gl16, m, dense_xs, Wd, bd[None, :], Wdl[:, 0][None, :],
                     a, r, w1a, w1b, b1p, w2p, b2p, w3p, c0)


# speedup vs baseline: 1.7604x; 1.7604x over previous
import functools

import jax
import jax.numpy as jnp
from jax import lax
from jax.experimental import pallas as pl
from jax.experimental.pallas import tpu as pltpu
from jax.experimental.pallas import tpu_sc as plsc

B = 4096
F = 26
V = 100000
D = 16
DENSE = 13
NF = F + 1
FD = F * D
EPS = 1e-5

NC = 2
NS = 16
NW = NC * NS
ROWS = B * F
RPW = ROWS // NW
CH = 128
NCH = RPW // CH


def _sc_lin_body(flatq_ref, lin16_ref, lout_ref, qidx_v, l16_v, sem_l):
    wid = lax.axis_index("s") * NC + lax.axis_index("c")
    r0 = wid * RPW
    pltpu.sync_copy(flatq_ref.at[pl.ds(r0, RPW)], qidx_v)

    def fire(j, carry):
        o = j * CH
        pltpu.async_copy(lin16_ref.at[qidx_v.at[pl.ds(o, CH)]],
                         l16_v.at[pl.ds(o, CH)], sem_l)
        return carry

    lax.fori_loop(0, NCH, fire, 0)

    def drain(j, carry):
        o = j * CH
        pltpu.make_async_copy(lin16_ref.at[qidx_v.at[pl.ds(o, CH)]],
                              l16_v.at[pl.ds(o, CH)], sem_l).wait()
        return carry

    lax.fori_loop(0, NCH, drain, 0)
    pltpu.sync_copy(l16_v, lout_ref.at[pl.ds(r0, RPW)])


def _sc_lin(flatq, lin16):
    mesh = plsc.VectorSubcoreMesh(core_axis_name="c", subcore_axis_name="s")
    k = pl.kernel(
        _sc_lin_body,
        out_type=jax.ShapeDtypeStruct((ROWS, D), jnp.float32),
        mesh=mesh,
        compiler_params=pltpu.CompilerParams(use_tc_tiling_on_sc=False),
        scratch_types=[
            pltpu.VMEM((RPW,), jnp.int32),
            pltpu.VMEM((RPW, D), jnp.float32),
            pltpu.SemaphoreType.DMA,
        ],
    )
    return k(flatq, lin16)


def _tr_body(x_ref, o_ref):
    o_ref[:, pl.ds(0, D)] = x_ref[...].T


TRC = 16384


def _tc_transpose(embT):
    return pl.pallas_call(
        _tr_body,
        grid=(pl.cdiv(F * V, TRC),),
        in_specs=[pl.BlockSpec((D, TRC), lambda i: (0, i))],
        out_specs=pl.BlockSpec((TRC, 128), lambda i: (i, 0)),
        out_shape=jax.ShapeDtypeStruct((F * V, 128), jnp.float32),
    )(embT)


def _sc_emb_body(q_ref, emb_ref, eout_ref, q_v, rows_v, sem_e):
    wid = lax.axis_index("s") * NC + lax.axis_index("c")
    r0 = wid * RPW

    pltpu.sync_copy(q_ref.at[pl.ds(r0, RPW)], q_v)

    def fire(j, buf):
        pltpu.async_copy(emb_ref.at[q_v.at[pl.ds(j * CH, CH)]],
                         rows_v.at[buf], sem_e)

    fire(0, 0)

    def chunk(j, carry):
        buf = lax.rem(j, 2)

        @pl.when(j + 1 < NCH)
        def _():
            fire(j + 1, 1 - buf)

        pltpu.make_async_copy(emb_ref.at[q_v.at[pl.ds(j * CH, CH)]],
                              rows_v.at[buf], sem_e).wait()
        pltpu.sync_copy(rows_v.at[buf],
                        eout_ref.at[pl.ds(r0 + j * CH, CH)])
        return carry

    lax.fori_loop(0, NCH, chunk, 0)


def _sc_emb(flat, embw):
    mesh = plsc.VectorSubcoreMesh(core_axis_name="c", subcore_axis_name="s")
    k = pl.kernel(
        _sc_emb_body,
        out_type=jax.ShapeDtypeStruct((ROWS, 128), jnp.float32),
        mesh=mesh,
        compiler_params=pltpu.CompilerParams(use_tc_tiling_on_sc=True),
        scratch_types=[
            pltpu.VMEM((RPW,), jnp.int32),
            pltpu.VMEM((2, CH, 128), jnp.float32),
            pltpu.SemaphoreType.DMA,
        ],
    )
    return k(flat, embw)


BT = 256


def _tc_body(ge_ref, gl16_ref, m_ref, dx_ref, wd_ref, bd_ref, wdl_ref, a_ref,
             r_ref, w1a_ref, w1b_ref, b1_ref, w2_ref, b2_ref, w3_ref, c0_ref,
             out_ref):
    ge = ge_ref[...]
    dx = dx_ref[...]
    de = jnp.dot(dx, wd_ref[...], preferred_element_type=jnp.float32) + bd_ref[...]
    s = jnp.dot(ge, a_ref[...], preferred_element_type=jnp.float32) + de
    sumsq = (jnp.sum(ge * ge, axis=1, keepdims=True)
             + jnp.sum(de * de, axis=1, keepdims=True))
    order2 = jnp.sum(s * s, axis=1, keepdims=True) - sumsq
    m_exp = jnp.dot(m_ref[...], r_ref[...], preferred_element_type=jnp.float32)
    lane = jnp.bitwise_and(
        lax.broadcasted_iota(jnp.int32, (BT, FD), 1), 15).astype(jnp.float32)
    lsel = jnp.where(m_exp == lane, gl16_ref[...], 0.0)
    order1 = (jnp.sum(lsel, axis=1, keepdims=True)
              + jnp.sum(dx * wdl_ref[...], axis=1, keepdims=True))
    h = (jnp.dot(ge, w1a_ref[...], preferred_element_type=jnp.float32)
         + jnp.dot(de, w1b_ref[...], preferred_element_type=jnp.float32)
         + b1_ref[...])
    h = jnp.where(h >= 0, h, 0.2 * h)
    h = jnp.dot(h, w2_ref[...], preferred_element_type=jnp.float32) + b2_ref[...]
    h = jnp.where(h >= 0, h, 0.2 * h)
    z = jnp.sum(h * w3_ref[...], axis=1, keepdims=True)
    out_ref[...] = z + order1 + 0.5 * order2 + c0_ref[...]


def _tc_dense(ge, gl16, m, dx, wd, bd, wdl, a, r, w1a, w1b, b1p, w2p, b2p,
              w3p, c0):
    full = lambda shape: pl.BlockSpec(shape, lambda i: (0,) * len(shape))
    return pl.pallas_call(
        _tc_body,
        grid=(B // BT,),
        in_specs=[
            pl.BlockSpec((BT, FD), lambda i: (i, 0)),
            pl.BlockSpec((BT, FD), lambda i: (i, 0)),
            pl.BlockSpec((BT, F), lambda i: (i, 0)),
            pl.BlockSpec((BT, DENSE), lambda i: (i, 0)),
            full((DENSE, D)),
            full((1, D)),
            full((1, DENSE)),
            full((FD, D)),
            full((F, FD)),
            full((FD, 256)),
            full((D, 256)),
            full((1, 256)),
            full((256, 128)),
            full((1, 128)),
            full((1, 128)),
            full((1, 1)),
        ],
        out_specs=pl.BlockSpec((BT, 1), lambda i: (i, 0)),
        out_shape=jax.ShapeDtypeStruct((B, 1), jnp.float32),
    )(ge, gl16, m, dx, wd, bd, wdl, a, r, w1a, w1b, b1p, w2p, b2p, w3p, c0)


def kernel(ids, sparse_xs, dense_xs, emb_table, lin_table, Wd, bd, Wdl, bdl,
           bn0_g, bn0_b, W1, b1, bn1_g, bn1_b, W2, b2, bn2_g, bn2_b, W3, b3,
           global_bias):
    offsets = jnp.arange(F, dtype=ids.dtype) * V
    flat = (ids + offsets[None, :]).reshape(ROWS).astype(jnp.int32)
    flatq = flat >> 4
    lin16 = lin_table.reshape(F * V // 16, 16)

    inv = 1.0 / jnp.sqrt(1.0 + EPS)
    f_idx = jnp.arange(FD) // D
    d_idx = jnp.arange(FD) % D
    perm_sparse = d_idx * NF + f_idx
    perm_dense = jnp.arange(D) * NF + F
    w1s = (bn0_g * inv)[:, None] * W1
    w1a = w1s[perm_sparse]
    w1b = w1s[perm_dense]
    b1p = (b1 + bn0_b @ W1)[None, :]
    w2p = (bn1_g * inv)[:, None] * W2
    b2p = (b2 + bn1_b @ W2)[None, :]
    w3p = ((bn2_g * inv) * W3[:, 0])[None, :]
    c0 = (b3[0] + bn2_b @ W3[:, 0] + global_bias[0] + bdl[0]).reshape(1, 1)
    a = jnp.tile(jnp.eye(D, dtype=jnp.float32), (F, 1))
    r = jnp.repeat(jnp.eye(F, dtype=jnp.float32), D, axis=1)
    m = jnp.bitwise_and(flat, 15).astype(jnp.float32).reshape(B, F)

    embw = _tc_transpose(emb_table.T)

    erows = _sc_emb(flat, embw)
    lrows = _sc_lin(flatq, lin16)
    ge = erows[:, :D].reshape(B, FD)
    gl16 = lrows.reshape(B, FD)

    return _tc_dense(ge, gl16, m, dense_xs, Wd, bd[None, :], Wdl[:, 0][None, :],
                     a, r, w1a, w1b, b1p, w2p, b2p, w3p, c0)
